# Initial kernel scaffold; baseline (speedup 1.0000x reference)
import jax
import jax.numpy as jnp
from jax.experimental import pallas as pl


def kernel(x, W):
    raise NotImplementedError("write your pallas kernel here")


# speedup vs baseline: 1.4617x; 1.4617x over previous
import functools

import jax
import jax.numpy as jnp
from jax import lax
from jax.experimental import pallas as pl
from jax.experimental.pallas import tpu as pltpu

B, C, H, W_DIM = 4, 384, 224, 224
K_TOP = C // 2
CB = 16
NCB = C // CB


def _score_body(x_ref, w_ref, o_ref):
    xb = x_ref[0]
    wb = w_ref[...]
    s1 = jnp.sum(xb, axis=1)
    total = jnp.sum(s1, axis=1)
    cl = s1[:, 0]
    cr = s1[:, W_DIM - 1]
    r0 = xb[:, 0, :]
    rb = xb[:, H - 1, :]
    rt_s = jnp.sum(r0, axis=1)
    rb_s = jnp.sum(rb, axis=1)
    x_tl = r0[:, 0]
    x_tr = r0[:, W_DIM - 1]
    x_bl = rb[:, 0]
    x_br = rb[:, W_DIM - 1]
    w_r0 = wb[:, 0] + wb[:, 1] + wb[:, 2]
    w_r2 = wb[:, 6] + wb[:, 7] + wb[:, 8]
    w_c0 = wb[:, 0] + wb[:, 3] + wb[:, 6]
    w_c2 = wb[:, 2] + wb[:, 5] + wb[:, 8]
    w_all = w_r0 + w_r2 + wb[:, 3] + wb[:, 4] + wb[:, 5]
    g = (
        total * w_all
        - rb_s * w_r0
        - rt_s * w_r2
        - cr * w_c0
        - cl * w_c2
        + x_br * wb[:, 0]
        + x_bl * wb[:, 2]
        + x_tr * wb[:, 6]
        + x_tl * wb[:, 8]
    ) * (1.0 / (H * W_DIM))
    o_ref[0, 0, 0, :] = g


def _gate_body(g_ref, o_ref):
    g = g_ref[...]
    gi = g[:, :, None]
    gj = g[:, None, :]
    ii = lax.broadcasted_iota(jnp.int32, (B, C, C), 1)
    jj = lax.broadcasted_iota(jnp.int32, (B, C, C), 2)
    beats = (gj > gi) | ((gj == gi) & (jj < ii))
    rank = jnp.sum(beats.astype(jnp.int32), axis=2)
    sel = rank < K_TOP
    gmax = jnp.max(g, axis=1, keepdims=True)
    e = jnp.where(sel, jnp.exp(g - gmax), 0.0)
    o_ref[...] = e / jnp.sum(e, axis=1, keepdims=True)


def _scale_body(w_smem, x_ref, o_ref):
    b = pl.program_id(0)
    c = pl.program_id(1)
    for i in range(CB):
        o_ref[0, i] = x_ref[0, i] * w_smem[b, c * CB + i]


@jax.jit
def kernel(x, W):
    w_taps = W.reshape(C, 9)
    g4 = pl.pallas_call(
        _score_body,
        grid=(B, NCB),
        in_specs=[
            pl.BlockSpec((1, CB, H, W_DIM), lambda b, c: (b, c, 0, 0)),
            pl.BlockSpec((CB, 9), lambda b, c: (c, 0)),
        ],
        out_specs=pl.BlockSpec((1, 1, 1, CB), lambda b, c: (b, c, 0, 0)),
        out_shape=jax.ShapeDtypeStruct((B, NCB, 1, CB), jnp.float32),
    )(x, w_taps)
    g = g4.reshape(B, C)
    w_dense = pl.pallas_call(
        _gate_body,
        in_specs=[pl.BlockSpec((B, C), lambda: (0, 0))],
        out_specs=pl.BlockSpec((B, C), lambda: (0, 0)),
        out_shape=jax.ShapeDtypeStruct((B, C), jnp.float32),
    )(g)
    out = pl.pallas_call(
        _scale_body,
        grid=(B, NCB),
        in_specs=[
            pl.BlockSpec(memory_space=pltpu.SMEM),
            pl.BlockSpec((1, CB, H, W_DIM), lambda b, c: (b, c, 0, 0)),
        ],
        out_specs=pl.BlockSpec((1, CB, H, W_DIM), lambda b, c: (b, c, 0, 0)),
        out_shape=jax.ShapeDtypeStruct((B, C, H, W_DIM), jnp.float32),
    )(w_dense, x)
    return out
